# baseline (speedup 1.0000x reference)
import functools

import numpy as np
import jax
import jax.numpy as jnp
from jax import lax
from jax.experimental import pallas as pl
from jax.experimental.pallas import tpu as pltpu

_BATCH = 128
_BN = 4096

_KS0 = np.uint32(0)
_KS1 = np.uint32(42)
_KS2 = np.uint32(np.uint32(0x1BD11BDA) ^ _KS0 ^ _KS1)
_KS = (_KS0, _KS1, _KS2)
_ROT = ((13, 15, 26, 6), (17, 29, 16, 24))
_TINY = np.float32(np.finfo(np.float32).tiny)


def _gumbel_bits(p):
    x0 = jnp.zeros_like(p) + _KS[0]
    x1 = p + _KS[1]
    for g in range(5):
        for r in _ROT[g % 2]:
            x0 = x0 + x1
            x1 = (x1 << np.uint32(r)) | (x1 >> np.uint32(32 - r))
            x1 = x1 ^ x0
        x0 = x0 + _KS[(g + 1) % 3]
        x1 = x1 + _KS[(g + 2) % 3] + np.uint32(g + 1)
    bits = x0 ^ x1
    fl = lax.bitcast_convert_type(
        (bits >> np.uint32(9)) | np.uint32(0x3F800000), jnp.float32
    ) - np.float32(1.0)
    u = jnp.maximum(_TINY, fl * (np.float32(1.0) - _TINY) + _TINY)
    return -jnp.log(-jnp.log(u))


def _body(nj, n, x_ref, w_ref, b_ref, logits_ref, act_ref,
          best_val, best_idx):
    j = pl.program_id(0)
    logits = (
        jnp.dot(x_ref[...], w_ref[...], preferred_element_type=jnp.float32)
        + b_ref[...]
    )
    logits_ref[...] = logits

    row = lax.broadcasted_iota(jnp.int32, logits.shape, 0)
    col = j * _BN + lax.broadcasted_iota(jnp.int32, logits.shape, 1)
    p = (row * n + col).astype(jnp.uint32)
    g = _gumbel_bits(p)
    valid = col < n
    score = jnp.where(valid, logits + g, -jnp.inf)
    blk_max = jnp.max(score, axis=1, keepdims=True)
    blk_arg = jnp.min(
        jnp.where(score == blk_max, col, jnp.iinfo(jnp.int32).max),
        axis=1, keepdims=True,
    )

    @pl.when(j == 0)
    def _():
        best_val[...] = jnp.full_like(best_val, -jnp.inf)
        best_idx[...] = jnp.zeros_like(best_idx)

    take = blk_max > best_val[...]
    best_val[...] = jnp.where(take, blk_max, best_val[...])
    best_idx[...] = jnp.where(take, blk_arg, best_idx[...])

    @pl.when(j == nj - 1)
    def _():
        act_ref[...] = best_idx[...]


def kernel(lstm_output, W, b):
    n = W.shape[1]
    nj = pl.cdiv(n, _BN)
    b2 = b.reshape(1, n)

    logits, action = pl.pallas_call(
        functools.partial(_body, nj, n),
        grid=(nj,),
        in_specs=[
            pl.BlockSpec((_BATCH, 128), lambda j: (0, 0)),
            pl.BlockSpec((128, _BN), lambda j: (0, j)),
            pl.BlockSpec((1, _BN), lambda j: (0, j)),
        ],
        out_specs=[
            pl.BlockSpec((_BATCH, _BN), lambda j: (0, j)),
            pl.BlockSpec((_BATCH, 1), lambda j: (0, 0)),
        ],
        out_shape=[
            jax.ShapeDtypeStruct((_BATCH, n), jnp.float32),
            jax.ShapeDtypeStruct((_BATCH, 1), jnp.int32),
        ],
        scratch_shapes=[
            pltpu.VMEM((_BATCH, 1), jnp.float32),
            pltpu.VMEM((_BATCH, 1), jnp.int32),
        ],
        compiler_params=pltpu.CompilerParams(
            dimension_semantics=("arbitrary",),
        ),
    )(lstm_output, W, b2)
    return (logits, action)


# speedup vs baseline: 1.0003x; 1.0003x over previous
import functools

import numpy as np
import jax
import jax.numpy as jnp
from jax import lax
from jax.experimental import pallas as pl
from jax.experimental.pallas import tpu as pltpu

_BATCH = 128
_BN = 4096

_KS0 = np.uint32(0)
_KS1 = np.uint32(42)
_KS2 = np.uint32(np.uint32(0x1BD11BDA) ^ _KS0 ^ _KS1)
_KS = (_KS0, _KS1, _KS2)
_ROT = ((13, 15, 26, 6), (17, 29, 16, 24))
_TINY = np.float32(np.finfo(np.float32).tiny)


def _gumbel_bits(p):
    x0 = jnp.zeros_like(p) + _KS[0]
    x1 = p + _KS[1]
    for g in range(5):
        for r in _ROT[g % 2]:
            x0 = x0 + x1
            x1 = (x1 << np.uint32(r)) | (x1 >> np.uint32(32 - r))
            x1 = x1 ^ x0
        x0 = x0 + _KS[(g + 1) % 3]
        x1 = x1 + _KS[(g + 2) % 3] + np.uint32(g + 1)
    bits = x0 ^ x1
    fl = lax.bitcast_convert_type(
        (bits >> np.uint32(9)) | np.uint32(0x3F800000), jnp.float32
    ) - np.float32(1.0)
    u = jnp.maximum(_TINY, fl * (np.float32(1.0) - _TINY) + _TINY)
    return -jnp.log(-jnp.log(u))


def _body(nj, n, x_ref, w_ref, b_ref, logits_ref, act_ref,
          best_val, best_idx):
    j = pl.program_id(0)
    shape = (x_ref.shape[0], w_ref.shape[1])
    row = lax.broadcasted_iota(jnp.int32, shape, 0)
    col = j * _BN + lax.broadcasted_iota(jnp.int32, shape, 1)
    p = (row * n + col).astype(jnp.uint32)
    g = _gumbel_bits(p)
    valid = col < n

    logits = (
        jnp.dot(x_ref[...], w_ref[...], preferred_element_type=jnp.float32)
        + b_ref[...]
    )
    logits_ref[...] = logits
    score = jnp.where(valid, logits + g, -jnp.inf)
    blk_max = jnp.max(score, axis=1, keepdims=True)
    blk_arg = jnp.min(
        jnp.where(score == blk_max, col, jnp.iinfo(jnp.int32).max),
        axis=1, keepdims=True,
    )

    @pl.when(j == 0)
    def _():
        best_val[...] = jnp.full_like(best_val, -jnp.inf)
        best_idx[...] = jnp.zeros_like(best_idx)

    take = blk_max > best_val[...]
    best_val[...] = jnp.where(take, blk_max, best_val[...])
    best_idx[...] = jnp.where(take, blk_arg, best_idx[...])

    @pl.when(j == nj - 1)
    def _():
        act_ref[...] = best_idx[...]


def kernel(lstm_output, W, b):
    n = W.shape[1]
    nj = pl.cdiv(n, _BN)
    b2 = b.reshape(1, n)

    logits, action = pl.pallas_call(
        functools.partial(_body, nj, n),
        grid=(nj,),
        in_specs=[
            pl.BlockSpec((_BATCH, 128), lambda j: (0, 0)),
            pl.BlockSpec((128, _BN), lambda j: (0, j)),
            pl.BlockSpec((1, _BN), lambda j: (0, j)),
        ],
        out_specs=[
            pl.BlockSpec((_BATCH, _BN), lambda j: (0, j)),
            pl.BlockSpec((_BATCH, 1), lambda j: (0, 0)),
        ],
        out_shape=[
            jax.ShapeDtypeStruct((_BATCH, n), jnp.float32),
            jax.ShapeDtypeStruct((_BATCH, 1), jnp.int32),
        ],
        scratch_shapes=[
            pltpu.VMEM((_BATCH, 1), jnp.float32),
            pltpu.VMEM((_BATCH, 1), jnp.int32),
        ],
        compiler_params=pltpu.CompilerParams(
            dimension_semantics=("arbitrary",),
        ),
    )(lstm_output, W, b2)
    return (logits, action)
